# baseline (speedup 1.0000x reference)
import functools

import jax
import jax.numpy as jnp
from jax import lax
from jax.experimental import pallas as pl
from jax.experimental.pallas import tpu as pltpu
from jax.experimental.pallas import tpu_sc as plsc

R, C = 64, 8192
NS, L = 16, 16
SC_ROWS = 8
HC = C // 2
TB = 8
NTB = (R - SC_ROWS) // TB
NCHUNK = 2
CHUNK = HC // NCHUNK
NBLK = CHUNK // L
U = 4
INT_MAX = 2**31 - 1

_sc_mesh = plsc.VectorSubcoreMesh(
    core_axis_name="c", subcore_axis_name="s", num_cores=1
)


@functools.partial(
    pl.kernel,
    out_type=[
        jax.ShapeDtypeStruct((NS, L), jnp.float32),
        jax.ShapeDtypeStruct((NS, L), jnp.int32),
    ],
    mesh=_sc_mesh,
    scratch_types=[
        pltpu.VMEM((HC,), jnp.float32),
        pltpu.VMEM((L,), jnp.float32),
        pltpu.VMEM((L,), jnp.int32),
        [pltpu.SemaphoreType.DMA] * NCHUNK,
    ],
)
def _sc_part_argmin(a_hbm, vals_out, idxs_out, buf, vmin_ref, vidx_ref, sems):
    s = lax.axis_index("s")
    row = s // 2
    col0 = (s % 2) * HC
    base = row * C + col0

    copies = [
        pltpu.async_copy(
            a_hbm.at[row, pl.ds(col0 + k * CHUNK, CHUNK)],
            buf.at[pl.ds(k * CHUNK, CHUNK)],
            sems[k],
        )
        for k in range(NCHUNK)
    ]

    lane = lax.iota(jnp.int32, L)
    inf = jnp.float32(jnp.inf)
    vmins = [jnp.full((L,), inf, jnp.float32) for _ in range(U)]
    vidxs = [jnp.zeros((L,), jnp.int32) for _ in range(U)]

    for k in range(NCHUNK):
        copies[k].wait()
        cbase = k * CHUNK

        init = tuple(vmins) + tuple(
            base + cbase + u * L + lane for u in range(U)
        ) + tuple(vidxs)

        @plsc.parallel_loop(0, NBLK // U, carry=init, unroll=2)
        def body(i, carry):
            vm = list(carry[:U])
            cur = list(carry[U : 2 * U])
            vi = list(carry[2 * U :])
            for u in range(U):
                v = buf[pl.ds(cbase + (i * U + u) * L, L)]
                upd = v < vm[u]
                vm[u] = jnp.where(upd, v, vm[u])
                vi[u] = jnp.where(upd, cur[u], vi[u])
                cur[u] = cur[u] + U * L
            return tuple(vm) + tuple(cur) + tuple(vi)

        out_carry = body
        vmins = list(out_carry[:U])
        vidxs = list(out_carry[2 * U :])

    vmin, vidx = vmins[0], vidxs[0]
    for u in range(1, U):
        upd = (vmins[u] < vmin) | ((vmins[u] == vmin) & (vidxs[u] < vidx))
        vmin = jnp.where(upd, vmins[u], vmin)
        vidx = jnp.where(upd, vidxs[u], vidx)

    vmin_ref[...] = vmin
    vidx_ref[...] = vidx
    pltpu.sync_copy(vmin_ref, vals_out.at[s])
    pltpu.sync_copy(vidx_ref, idxs_out.at[s])


def _tc_scan_body(a_ref, val_ref, idx_ref, mcar, icar):
    i = pl.program_id(0)
    v = a_ref[...]
    m = jnp.min(v)

    @pl.when(i == 0)
    def _():
        mcar[0] = jnp.float32(jnp.inf)
        icar[0] = jnp.int32(INT_MAX)

    @pl.when(m < mcar[0])
    def _():
        row = lax.broadcasted_iota(jnp.int32, (TB, C), 0)
        col = lax.broadcasted_iota(jnp.int32, (TB, C), 1)
        flat = (row + SC_ROWS + i * TB) * C + col
        mcar[0] = m
        icar[0] = jnp.min(jnp.where(v == m, flat, INT_MAX))

    @pl.when(i == NTB - 1)
    def _():
        val_ref[...] = jnp.full((1, 128), mcar[0], jnp.float32)
        idx_ref[...] = jnp.full((1, 128), icar[0], jnp.int32)


_tc_scan = pl.pallas_call(
    _tc_scan_body,
    grid=(NTB,),
    in_specs=[pl.BlockSpec((TB, C), lambda i: (i + SC_ROWS // TB, 0))],
    out_shape=[
        jax.ShapeDtypeStruct((1, 128), jnp.float32),
        jax.ShapeDtypeStruct((1, 128), jnp.int32),
    ],
    out_specs=[
        pl.BlockSpec((1, 128), lambda i: (0, 0)),
        pl.BlockSpec((1, 128), lambda i: (0, 0)),
    ],
    scratch_shapes=[
        pltpu.SMEM((1,), jnp.float32),
        pltpu.SMEM((1,), jnp.int32),
    ],
)


def _merge_body(vals_ref, idxs_ref, tcv_ref, tci_ref, out_ref):
    vals = vals_ref[...]
    idxs = idxs_ref[...]
    tcv = tcv_ref[...]
    tci = tci_ref[...]
    m = jnp.minimum(jnp.min(vals), jnp.min(tcv))
    sc_best = jnp.min(jnp.where(vals == m, idxs, INT_MAX))
    tc_best = jnp.min(jnp.where(tcv == m, tci, INT_MAX))
    out_ref[0, 0] = jnp.minimum(sc_best, tc_best)


_merge = pl.pallas_call(
    _merge_body,
    out_shape=jax.ShapeDtypeStruct((1, 1), jnp.int32),
    out_specs=pl.BlockSpec(memory_space=pltpu.SMEM),
)


def kernel(a):
    vals, idxs = _sc_part_argmin(a)
    tcv, tci = _tc_scan(a)
    out = _merge(vals, idxs, tcv, tci)
    return out[0, 0].astype(jnp.int64)


# speedup vs baseline: 1.0269x; 1.0269x over previous
import functools

import jax
import jax.numpy as jnp
from jax import lax
from jax.experimental import pallas as pl
from jax.experimental.pallas import tpu as pltpu
from jax.experimental.pallas import tpu_sc as plsc

R, C = 64, 8192
NS, L = 16, 16
SC_ROWS = 16
TC_ROWS = R - SC_ROWS
TB = 8
NTB = TC_ROWS // TB
NCHUNK = 2
CHUNK = C // NCHUNK
NBLK = CHUNK // L
U = 4
INT_MAX = 2**31 - 1

_sc_mesh = plsc.VectorSubcoreMesh(
    core_axis_name="c", subcore_axis_name="s", num_cores=1
)


@functools.partial(
    pl.kernel,
    out_type=[
        jax.ShapeDtypeStruct((NS, L), jnp.float32),
        jax.ShapeDtypeStruct((NS, L), jnp.int32),
    ],
    mesh=_sc_mesh,
    scratch_types=[
        pltpu.VMEM((C,), jnp.float32),
        pltpu.VMEM((L,), jnp.float32),
        pltpu.VMEM((L,), jnp.int32),
        [pltpu.SemaphoreType.DMA] * NCHUNK,
    ],
)
def _sc_part_argmin(a_hbm, vals_out, idxs_out, buf, vmin_ref, vidx_ref, sems):
    s = lax.axis_index("s")
    base = s * C

    copies = [
        pltpu.async_copy(
            a_hbm.at[s, pl.ds(k * CHUNK, CHUNK)],
            buf.at[pl.ds(k * CHUNK, CHUNK)],
            sems[k],
        )
        for k in range(NCHUNK)
    ]

    lane = lax.iota(jnp.int32, L)
    inf = jnp.float32(jnp.inf)
    vmins = [jnp.full((L,), inf, jnp.float32) for _ in range(U)]
    vidxs = [jnp.zeros((L,), jnp.int32) for _ in range(U)]

    for k in range(NCHUNK):
        copies[k].wait()
        cbase = k * CHUNK

        init = tuple(vmins) + tuple(
            base + cbase + u * L + lane for u in range(U)
        ) + tuple(vidxs)

        @plsc.parallel_loop(0, NBLK // U, carry=init, unroll=2)
        def body(i, carry):
            vm = list(carry[:U])
            cur = list(carry[U : 2 * U])
            vi = list(carry[2 * U :])
            for u in range(U):
                v = buf[pl.ds(cbase + (i * U + u) * L, L)]
                upd = v < vm[u]
                vm[u] = jnp.where(upd, v, vm[u])
                vi[u] = jnp.where(upd, cur[u], vi[u])
                cur[u] = cur[u] + U * L
            return tuple(vm) + tuple(cur) + tuple(vi)

        out_carry = body
        vmins = list(out_carry[:U])
        vidxs = list(out_carry[2 * U :])

    vmin, vidx = vmins[0], vidxs[0]
    for u in range(1, U):
        upd = (vmins[u] < vmin) | ((vmins[u] == vmin) & (vidxs[u] < vidx))
        vmin = jnp.where(upd, vmins[u], vmin)
        vidx = jnp.where(upd, vidxs[u], vidx)

    vmin_ref[...] = vmin
    vidx_ref[...] = vidx
    pltpu.sync_copy(vmin_ref, vals_out.at[s])
    pltpu.sync_copy(vidx_ref, idxs_out.at[s])


def _tc_scan_body(a_ref, val_ref, idx_ref, mcar, icar):
    i = pl.program_id(0)
    v = a_ref[...]
    m = jnp.min(v)
    row = lax.broadcasted_iota(jnp.int32, (TB, C), 0)
    col = lax.broadcasted_iota(jnp.int32, (TB, C), 1)
    flat = (row + SC_ROWS + i * TB) * C + col
    mi = jnp.min(jnp.where(v == m, flat, INT_MAX))

    @pl.when(i == 0)
    def _():
        mcar[0] = jnp.float32(jnp.inf)
        icar[0] = jnp.int32(INT_MAX)

    upd = m < mcar[0]
    mcar[0] = jnp.where(upd, m, mcar[0])
    icar[0] = jnp.where(upd, mi, icar[0])

    @pl.when(i == NTB - 1)
    def _():
        val_ref[...] = jnp.full((1, 128), mcar[0], jnp.float32)
        idx_ref[...] = jnp.full((1, 128), icar[0], jnp.int32)


_tc_scan = pl.pallas_call(
    _tc_scan_body,
    grid=(NTB,),
    in_specs=[pl.BlockSpec((TB, C), lambda i: (i + SC_ROWS // TB, 0))],
    out_shape=[
        jax.ShapeDtypeStruct((1, 128), jnp.float32),
        jax.ShapeDtypeStruct((1, 128), jnp.int32),
    ],
    out_specs=[
        pl.BlockSpec((1, 128), lambda i: (0, 0)),
        pl.BlockSpec((1, 128), lambda i: (0, 0)),
    ],
    scratch_shapes=[
        pltpu.SMEM((1,), jnp.float32),
        pltpu.SMEM((1,), jnp.int32),
    ],
)


def _merge_body(vals_ref, idxs_ref, tcv_ref, tci_ref, out_ref):
    vals = vals_ref[...]
    idxs = idxs_ref[...]
    tcv = tcv_ref[...]
    tci = tci_ref[...]
    m = jnp.minimum(jnp.min(vals), jnp.min(tcv))
    sc_best = jnp.min(jnp.where(vals == m, idxs, INT_MAX))
    tc_best = jnp.min(jnp.where(tcv == m, tci, INT_MAX))
    out_ref[0, 0] = jnp.minimum(sc_best, tc_best)


_merge = pl.pallas_call(
    _merge_body,
    out_shape=jax.ShapeDtypeStruct((1, 1), jnp.int32),
    out_specs=pl.BlockSpec(memory_space=pltpu.SMEM),
)


def kernel(a):
    vals, idxs = _sc_part_argmin(a)
    tcv, tci = _tc_scan(a)
    out = _merge(vals, idxs, tcv, tci)
    return out[0, 0].astype(jnp.int64)
